# Initial kernel scaffold; baseline (speedup 1.0000x reference)
import jax
import jax.numpy as jnp
from jax.experimental import pallas as pl


def kernel(x1, x2, params):
    raise NotImplementedError("write your pallas kernel here")


# speedup vs baseline: 2.0595x; 2.0595x over previous
import jax
import jax.numpy as jnp
from jax.experimental import pallas as pl
from jax.experimental.pallas import tpu as pltpu

DIM = 512
ATT = 256
BQ = 512


def _elu(x):
    return jnp.where(x > 0, x, jnp.exp(jnp.minimum(x, 0.0)) - 1.0)


def _rmsnorm(x, w, eps=1e-8):
    return x * w / jnp.sqrt(jnp.mean(x * x, axis=-1, keepdims=True) + eps)


def _dot(a, b):
    return jnp.dot(a, b, preferred_element_type=jnp.float32)


def _mome_kernel(x1_ref, x2_ref, wq_ref, wk_ref, wv_ref, wo_ref,
                 n1w_ref, n2w_ref, w1_ref, b1_ref, w2_ref, b2_ref,
                 milv_ref, milu_ref, milw_ref, pw_ref, pb_ref,
                 out_ref, kT_ref, v_ref, bias_ref):
    i = pl.program_id(1)

    @pl.when(i == 0)
    def _per_batch():
        x2 = x2_ref[0]
        kT_ref[...] = _dot(x2, wk_ref[...]).T
        v_ref[...] = _dot(x2, wv_ref[...])
        h2 = _elu(_dot(_rmsnorm(x2, n2w_ref[...]), w2_ref[...]) + b2_ref[...])
        snn2 = jnp.mean(h2, axis=0, keepdims=True)
        a = jnp.tanh(_dot(x2, milv_ref[...])) * jax.nn.sigmoid(_dot(x2, milu_ref[...]))
        scores = jnp.sum(a * milw_ref[...], axis=-1, keepdims=True)
        e = jnp.exp(scores - jnp.max(scores))
        att = e / jnp.sum(e)
        pooled = jnp.sum(att * x2, axis=0, keepdims=True)
        bias_ref[...] = snn2 + _dot(pooled, pw_ref[...]) + pb_ref[...]

    x1 = x1_ref[0]
    q = _dot(x1, wq_ref[...])
    s = _dot(q, kT_ref[...]) * (1.0 / jnp.sqrt(float(DIM)))
    s = s - jnp.max(s, axis=-1, keepdims=True)
    p = jnp.exp(s)
    p = p / jnp.sum(p, axis=-1, keepdims=True)
    coa = _dot(_dot(p, v_ref[...]), wo_ref[...])
    snn1 = _elu(_dot(_rmsnorm(x1, n1w_ref[...]), w1_ref[...]) + b1_ref[...])
    out_ref[0] = 3.0 * x1 + coa + snn1 + bias_ref[...]


def kernel(x1, x2, params):
    B, N1, _ = x1.shape
    N2 = x2.shape[1]
    p = params
    row = lambda v: v.reshape(1, -1)
    full2 = lambda a: pl.BlockSpec(a.shape, lambda b, i: (0, 0))

    weights = (p['coa_Wq'], p['coa_Wk'], p['coa_Wv'], p['coa_Wo'],
               row(p['norm1_w']), row(p['norm2_w']),
               p['snn1_W'], row(p['snn1_b']), p['snn2_W'], row(p['snn2_b']),
               p['mil_V'], p['mil_U'], row(p['mil_w'][:, 0]),
               p['mil_proj_W'], row(p['mil_proj_b']))

    out = pl.pallas_call(
        _mome_kernel,
        grid=(B, N1 // BQ),
        in_specs=[pl.BlockSpec((1, BQ, DIM), lambda b, i: (b, i, 0)),
                  pl.BlockSpec((1, N2, DIM), lambda b, i: (b, 0, 0))]
                 + [full2(w) for w in weights],
        out_specs=pl.BlockSpec((1, BQ, DIM), lambda b, i: (b, i, 0)),
        out_shape=jax.ShapeDtypeStruct((B, N1, DIM), jnp.float32),
        scratch_shapes=[pltpu.VMEM((DIM, N2), jnp.float32),
                        pltpu.VMEM((N2, DIM), jnp.float32),
                        pltpu.VMEM((1, DIM), jnp.float32)],
        compiler_params=pltpu.CompilerParams(
            dimension_semantics=("arbitrary", "arbitrary")),
    )(x1, x2, *weights)
    return (out, jnp.zeros((), jnp.float32), -1)
